# Initial kernel scaffold; baseline (speedup 1.0000x reference)
import functools

import jax
import jax.numpy as jnp
from jax import lax
from jax.experimental import pallas as pl
from jax.experimental.pallas import tpu as pltpu
from jax.experimental.pallas import tpu_sc as plsc

BATCH = 4096
SEQ_LEN = 200
HIDDEN = 64
LANES = 16

NUM_CORES = 2
NUM_SUBCORES = 16
NUM_WORKERS = NUM_CORES * NUM_SUBCORES
BPW = BATCH // NUM_WORKERS

IDX_CHUNKS = 2
IDX_PER_CHUNK = SEQ_LEN // IDX_CHUNKS


def _body(x_hbm, wt_hbm, pt_hbm, out_hbm, idx_v, rows_v, pos_v, sem):
    wid = lax.axis_index("s") * NUM_CORES + lax.axis_index("c")

    pltpu.sync_copy(pt_hbm.at[pl.ds(0, SEQ_LEN)], pos_v)

    def chunk_body(g, carry):
        row = wid * BPW + g
        pltpu.sync_copy(x_hbm.at[row], idx_v)
        cp0 = pltpu.async_copy(
            wt_hbm.at[idx_v.at[0]], rows_v.at[pl.ds(0, IDX_PER_CHUNK)], sem
        )
        cp1 = pltpu.async_copy(
            wt_hbm.at[idx_v.at[1]],
            rows_v.at[pl.ds(IDX_PER_CHUNK, IDX_PER_CHUNK)],
            sem,
        )
        cp0.wait()
        cp1.wait()

        def add_body(j, c):
            for h in range(HIDDEN // LANES):
                vec = pos_v[j, pl.ds(h * LANES, LANES)]
                plsc.addupdate(rows_v.at[j, pl.ds(h * LANES, LANES)], vec)
            return c

        lax.fori_loop(0, SEQ_LEN, add_body, 0)
        pltpu.sync_copy(rows_v, out_hbm.at[row])
        return carry

    lax.fori_loop(0, BPW, chunk_body, 0)


@jax.jit
def _run(x_r, word_table, pos_table):
    mesh = plsc.VectorSubcoreMesh(core_axis_name="c", subcore_axis_name="s")
    return pl.kernel(
        _body,
        out_type=jax.ShapeDtypeStruct((BATCH, SEQ_LEN, HIDDEN), jnp.float32),
        mesh=mesh,
        scratch_types=[
            pltpu.VMEM((IDX_CHUNKS, IDX_PER_CHUNK), jnp.int32),
            pltpu.VMEM((SEQ_LEN, HIDDEN), jnp.float32),
            pltpu.VMEM((SEQ_LEN, HIDDEN), jnp.float32),
            pltpu.SemaphoreType.DMA,
        ],
    )(x_r, word_table, pos_table)


def kernel(x, word_table, pos_table):
    x_r = x.astype(jnp.int32).reshape(BATCH, IDX_CHUNKS, IDX_PER_CHUNK)
    return _run(x_r, word_table, pos_table)


# speedup vs baseline: 2.3624x; 2.3624x over previous
import functools

import jax
import jax.numpy as jnp
from jax import lax
from jax.experimental import pallas as pl
from jax.experimental.pallas import tpu as pltpu
from jax.experimental.pallas import tpu_sc as plsc

BATCH = 4096
SEQ_LEN = 200
HIDDEN = 64
LANES = 16

NUM_CORES = 2
NUM_SUBCORES = 16
NUM_WORKERS = NUM_CORES * NUM_SUBCORES
BPW = BATCH // NUM_WORKERS

IDX_CHUNKS = 2
IDX_PER_CHUNK = SEQ_LEN // IDX_CHUNKS


def _body(x_hbm, wt_hbm, pt_hbm, out_hbm, idx_v, rows_v, pos_v, sem):
    wid = lax.axis_index("s") * NUM_CORES + lax.axis_index("c")

    pltpu.sync_copy(pt_hbm.at[pl.ds(0, SEQ_LEN)], pos_v)

    def chunk_body(g, carry):
        row = wid * BPW + g
        pltpu.sync_copy(x_hbm.at[row], idx_v)
        cp0 = pltpu.async_copy(
            wt_hbm.at[idx_v.at[0]], rows_v.at[pl.ds(0, IDX_PER_CHUNK)], sem
        )
        cp1 = pltpu.async_copy(
            wt_hbm.at[idx_v.at[1]],
            rows_v.at[pl.ds(IDX_PER_CHUNK, IDX_PER_CHUNK)],
            sem,
        )
        cp0.wait()
        cp1.wait()

        def add_body(j, c):
            for h in range(HIDDEN // LANES):
                vec = pos_v[j, pl.ds(h * LANES, LANES)]
                plsc.addupdate(rows_v.at[j, pl.ds(h * LANES, LANES)], vec)
            return c

        lax.fori_loop(0, SEQ_LEN, add_body, 0)
        pltpu.sync_copy(rows_v, out_hbm.at[row])
        return carry

    lax.fori_loop(0, BPW, chunk_body, 0)


@jax.jit
def _run(x_r, word_table, pos_table):
    mesh = plsc.VectorSubcoreMesh(core_axis_name="c", subcore_axis_name="s")
    return pl.kernel(
        _body,
        out_type=jax.ShapeDtypeStruct((BATCH, SEQ_LEN, HIDDEN), jnp.float32),
        mesh=mesh,
        compiler_params=pltpu.CompilerParams(use_tc_tiling_on_sc=False),
        scratch_types=[
            pltpu.VMEM((IDX_CHUNKS, IDX_PER_CHUNK), jnp.int32),
            pltpu.VMEM((SEQ_LEN, HIDDEN), jnp.float32),
            pltpu.VMEM((SEQ_LEN, HIDDEN), jnp.float32),
            pltpu.SemaphoreType.DMA,
        ],
    )(x_r, word_table, pos_table)


def kernel(x, word_table, pos_table):
    x_r = x.astype(jnp.int32).reshape(BATCH, IDX_CHUNKS, IDX_PER_CHUNK)
    return _run(x_r, word_table, pos_table)


# speedup vs baseline: 2.7344x; 1.1574x over previous
import jax
import jax.numpy as jnp
from jax import lax
from jax.experimental import pallas as pl
from jax.experimental.pallas import tpu as pltpu
from jax.experimental.pallas import tpu_sc as plsc

BATCH = 4096
SEQ_LEN = 200
HIDDEN = 64
LANES = 16

NUM_CORES = 2
NUM_SUBCORES = 16
NUM_WORKERS = NUM_CORES * NUM_SUBCORES
BPW = BATCH // NUM_WORKERS

IDX_CHUNKS = 2
IDX_PER_CHUNK = SEQ_LEN // IDX_CHUNKS

NBUF = 4
SUPERS = BPW // NBUF


def _body(x_hbm, wt_hbm, pt_hbm, out_hbm, idx_all, pos_v, rows, sems_g, sems_o):
    wid = lax.axis_index("s") * NUM_CORES + lax.axis_index("c")
    base = wid * BPW

    pltpu.sync_copy(x_hbm.at[pl.ds(base, BPW)], idx_all)
    pltpu.sync_copy(pt_hbm.at[pl.ds(0, SEQ_LEN)], pos_v)

    def issue_gather(c, b):
        for j in range(IDX_CHUNKS):
            pltpu.async_copy(
                wt_hbm.at[idx_all.at[c, j]],
                rows[b].at[pl.ds(j * IDX_PER_CHUNK, IDX_PER_CHUNK)],
                sems_g[b],
            )

    def wait_gather(b):
        pltpu.make_async_copy(wt_hbm.at[pl.ds(0, SEQ_LEN)], rows[b], sems_g[b]).wait()

    def wait_out(b):
        pltpu.make_async_copy(rows[b], out_hbm.at[0], sems_o[b]).wait()

    def add_pos(b):
        def add_body(j, c):
            for h in range(HIDDEN // LANES):
                vec = pos_v[j, pl.ds(h * LANES, LANES)]
                plsc.addupdate(rows[b].at[j, pl.ds(h * LANES, LANES)], vec)
            return c

        lax.fori_loop(0, SEQ_LEN, add_body, 0)

    for b in range(NBUF):
        issue_gather(b, b)

    def super_body(s, carry):
        c0 = s * NBUF
        for b in range(NBUF):
            wait_gather(b)
            add_pos(b)
            pltpu.async_copy(rows[b], out_hbm.at[base + c0 + b], sems_o[b])
        for b in range(NBUF):
            wait_out(b)
            issue_gather(c0 + NBUF + b, b)
        return carry

    lax.fori_loop(0, SUPERS - 1, super_body, 0)

    c0 = (SUPERS - 1) * NBUF
    for b in range(NBUF):
        wait_gather(b)
        add_pos(b)
        pltpu.async_copy(rows[b], out_hbm.at[base + c0 + b], sems_o[b])
    for b in range(NBUF):
        wait_out(b)


@jax.jit
def _run(x_r, word_table, pos_table):
    mesh = plsc.VectorSubcoreMesh(core_axis_name="c", subcore_axis_name="s")
    return pl.kernel(
        _body,
        out_type=jax.ShapeDtypeStruct((BATCH, SEQ_LEN, HIDDEN), jnp.float32),
        mesh=mesh,
        compiler_params=pltpu.CompilerParams(use_tc_tiling_on_sc=False),
        scratch_types=[
            pltpu.VMEM((BPW, IDX_CHUNKS, IDX_PER_CHUNK), jnp.int32),
            pltpu.VMEM((SEQ_LEN, HIDDEN), jnp.float32),
            [pltpu.VMEM((SEQ_LEN, HIDDEN), jnp.float32) for _ in range(NBUF)],
            [pltpu.SemaphoreType.DMA for _ in range(NBUF)],
            [pltpu.SemaphoreType.DMA for _ in range(NBUF)],
        ],
    )(x_r, word_table, pos_table)


def kernel(x, word_table, pos_table):
    x_r = x.astype(jnp.int32).reshape(BATCH, IDX_CHUNKS, IDX_PER_CHUNK)
    return _run(x_r, word_table, pos_table)
